# baseline (speedup 1.0000x reference)
import functools
import math

import jax
import jax.numpy as jnp
from jax import lax
from jax.experimental import pallas as pl
from jax.experimental.pallas import tpu as pltpu
from jax.experimental.pallas import tpu_sc as plsc

D_MODEL = 64
SCALE = math.sqrt(D_MODEL)
NC, NS, L = 2, 16, 16
NW = NC * NS

IDX_W = 128
GATHERS = 4
CHUNK = IDX_W * GATHERS


@functools.cache
def _make_emb(B, V):
    rows_w = B // NW
    idxrows_w = rows_w // IDX_W
    chunks_w = rows_w // CHUNK
    mesh = plsc.VectorSubcoreMesh(core_axis_name="c", subcore_axis_name="s")

    @functools.partial(
        pl.kernel,
        out_type=jax.ShapeDtypeStruct((B, D_MODEL), jnp.float32),
        mesh=mesh,
        scratch_types=[
            pltpu.VMEM((idxrows_w, IDX_W), jnp.int32),
            pltpu.VMEM((CHUNK, D_MODEL), jnp.float32),
            pltpu.VMEM((CHUNK,), jnp.float32),
            pltpu.SemaphoreType.DMA,
        ],
        compiler_params=pltpu.CompilerParams(
            use_tc_tiling_on_sc=False, needs_layout_passes=False
        ),
    )
    def emb(lut_hbm, idx_hbm, out_hbm, idx_v, buf, sbuf, gsem):
        wid = lax.axis_index("s") * NC + lax.axis_index("c")
        idx_row0 = wid * idxrows_w
        out_row0 = wid * rows_w
        pltpu.sync_copy(idx_hbm.at[pl.ds(idx_row0, idxrows_w)], idx_v)

        @pl.loop(0, chunks_w)
        def chunk_loop(i):
            cps = [
                pltpu.async_copy(
                    lut_hbm.at[idx_v.at[i * GATHERS + b]],
                    buf.at[pl.ds(b * IDX_W, IDX_W)],
                    gsem,
                )
                for b in range(GATHERS)
            ]
            for cp in cps:
                cp.wait()

            @pl.loop(0, GATHERS)
            def srow(jj):
                for gg in range(IDX_W // L):
                    iv = idx_v[i * GATHERS + jj, pl.ds(gg * L, L)]
                    sv = jnp.where(iv == 0, jnp.float32(0.0), jnp.float32(SCALE))
                    sbuf[pl.ds(jj * IDX_W + gg * L, L)] = sv

            @pl.loop(0, CHUNK, unroll=8)
            def scale_loop(r):
                sc = plsc.load_gather(sbuf, [jnp.full((L,), r, jnp.int32)])
                for cc in range(D_MODEL // L):
                    sl = buf[r, pl.ds(cc * L, L)]
                    buf[r, pl.ds(cc * L, L)] = sl * sc

            pltpu.sync_copy(buf, out_hbm.at[pl.ds(out_row0 + i * CHUNK, CHUNK)])

    return emb


def kernel(x, lut):
    s0, s1 = x.shape
    B = s0 * s1
    xf = x.reshape(-1).astype(jnp.int32).reshape(B // IDX_W, IDX_W)
    out = _make_emb(B, lut.shape[0])(lut, xf)
    return out.reshape(s0, s1, D_MODEL)


# speedup vs baseline: 1.0750x; 1.0750x over previous
import functools
import math

import jax
import jax.numpy as jnp
from jax import lax
from jax.experimental import pallas as pl
from jax.experimental.pallas import tpu as pltpu
from jax.experimental.pallas import tpu_sc as plsc

D_MODEL = 64
SCALE = math.sqrt(D_MODEL)
NC, NS, L = 2, 16, 16
NW = NC * NS

IDX_W = 128
GATHERS = 4
CHUNK = IDX_W * GATHERS
NBUF = 2


@functools.cache
def _make_emb(B, V):
    rows_w = B // NW
    idxrows_w = rows_w // IDX_W
    chunks_w = rows_w // CHUNK
    assert chunks_w % NBUF == 0
    mesh = plsc.VectorSubcoreMesh(core_axis_name="c", subcore_axis_name="s")

    @functools.partial(
        pl.kernel,
        out_type=jax.ShapeDtypeStruct((B, D_MODEL), jnp.float32),
        mesh=mesh,
        scratch_types=[
            pltpu.VMEM((idxrows_w, IDX_W), jnp.int32),
            pltpu.VMEM((NBUF, CHUNK, D_MODEL), jnp.float32),
            pltpu.VMEM((CHUNK,), jnp.float32),
            pltpu.SemaphoreType.DMA,
            pltpu.SemaphoreType.DMA,
        ],
        compiler_params=pltpu.CompilerParams(
            use_tc_tiling_on_sc=False, needs_layout_passes=False
        ),
    )
    def emb(lut_hbm, idx_hbm, out_hbm, idx_v, buf, sbuf, gsem0, gsem1):
        gsems = (gsem0, gsem1)
        wid = lax.axis_index("s") * NC + lax.axis_index("c")
        idx_row0 = wid * idxrows_w
        out_row0 = wid * rows_w
        pltpu.sync_copy(idx_hbm.at[pl.ds(idx_row0, idxrows_w)], idx_v)

        def gather_refs(ch, b, k):
            src = lut_hbm.at[idx_v.at[ch * GATHERS + k]]
            dst = buf.at[b].at[pl.ds(k * IDX_W, IDX_W)]
            return src, dst

        def fire(ch, b):
            for k in range(GATHERS):
                src, dst = gather_refs(ch, b, k)
                pltpu.async_copy(src, dst, gsems[b])

        def drain(ch, b):
            for k in range(GATHERS):
                src, dst = gather_refs(ch, b, k)
                pltpu.make_async_copy(src, dst, gsems[b]).wait()

        fire(0, 0)

        @pl.loop(0, chunks_w, step=NBUF)
        def outer(i):
            for b in range(NBUF):
                ch = i + b

                @pl.when(ch + 1 < chunks_w)
                def _():
                    fire(ch + 1, (b + 1) % NBUF)

                drain(ch, b)

                @pl.loop(0, GATHERS)
                def srow(jj):
                    for gg in range(IDX_W // L):
                        iv = idx_v[ch * GATHERS + jj, pl.ds(gg * L, L)]
                        sv = jnp.where(
                            iv == 0, jnp.float32(0.0), jnp.float32(SCALE)
                        )
                        sbuf[pl.ds(jj * IDX_W + gg * L, L)] = sv

                @pl.loop(0, CHUNK, unroll=8)
                def scale_loop(r):
                    sc = plsc.load_gather(sbuf, [jnp.full((L,), r, jnp.int32)])
                    for cc in range(D_MODEL // L):
                        sl = buf[b, r, pl.ds(cc * L, L)]
                        buf[b, r, pl.ds(cc * L, L)] = sl * sc

                pltpu.sync_copy(
                    buf.at[b], out_hbm.at[pl.ds(out_row0 + ch * CHUNK, CHUNK)]
                )

    return emb


def kernel(x, lut):
    s0, s1 = x.shape
    B = s0 * s1
    xf = x.reshape(-1).astype(jnp.int32).reshape(B // IDX_W, IDX_W)
    out = _make_emb(B, lut.shape[0])(lut, xf)
    return out.reshape(s0, s1, D_MODEL)
